# baseline (speedup 1.0000x reference)
import functools

import jax
import jax.numpy as jnp
from jax import lax
from jax.experimental import pallas as pl
from jax.experimental.pallas import tpu as pltpu
from jax.experimental.pallas import tpu_sc as plsc

OMEGA = 32
SINK = 4
P_RATIO = 0.1
R_RATIO = 0.3
START_IDX = 1

W_PAD = 64


def _reduce_body(attn_ref, col_ref):
    col_ref[...] = attn_ref[0].sum(axis=0)[None, None, :]


def _build_reduce_kernel(h_num, s_len):
    return pl.pallas_call(
        _reduce_body,
        grid=(h_num,),
        in_specs=[pl.BlockSpec((1, s_len, s_len), lambda h: (h, 0, 0))],
        out_specs=pl.BlockSpec((1, 1, s_len), lambda h: (h, 0, 0)),
        out_shape=jax.ShapeDtypeStruct((h_num, 1, s_len), jnp.float32),
        compiler_params=pltpu.CompilerParams(
            dimension_semantics=("arbitrary",)),
    )


def _index_body(h_num, s_len, idx_pad, n_eligible, k_windows, sink, omega,
                mid_end, local_off, col_ref, idx_ref):
    cs = col_ref[...].reshape(h_num, s_len)

    w3 = lax.broadcasted_iota(jnp.int32, (h_num, W_PAD, s_len), 1)
    s3 = lax.broadcasted_iota(jnp.int32, (h_num, W_PAD, s_len), 2)
    in_win = (s3 >= sink) & ((s3 - sink) // omega == w3) & (w3 < n_eligible)
    cs3 = jnp.broadcast_to(cs[:, None, :], (h_num, W_PAD, s_len))
    win = jnp.where(in_win, cs3, 0.0).sum(axis=2)

    l64 = lax.broadcasted_iota(jnp.int32, (h_num, W_PAD), 1)
    neg = jnp.float32(-jnp.inf)
    base = jnp.where(l64 < n_eligible, win, neg)

    def step(_, keep):
        cur = jnp.where(keep > 0, neg, base)
        m = jnp.max(cur, axis=1, keepdims=True)
        first = jnp.min(jnp.where(cur == m, l64, W_PAD), axis=1, keepdims=True)
        return jnp.where(l64 == first, 1, keep)

    km_i = lax.fori_loop(0, k_windows, step,
                         jnp.zeros((h_num, W_PAD), jnp.int32))
    km = km_i > 0

    wr = lax.broadcasted_iota(jnp.int32, (W_PAD, W_PAD), 0)
    wp = lax.broadcasted_iota(jnp.int32, (W_PAD, W_PAD), 1)
    le = (wr <= wp).astype(jnp.float32)
    cums = jax.lax.dot_general(km.astype(jnp.float32), le,
                               (((1,), (0,)), ((), ())),
                               precision=jax.lax.Precision.HIGHEST)
    pos = cums.astype(jnp.int32) - 1

    sl = lax.broadcasted_iota(jnp.int32, (1, idx_pad), 1)
    jm = (sl - sink) // omega
    rm = (sl - sink) % omega
    tok_mid = jnp.zeros((h_num, idx_pad), jnp.int32)
    for j in range(k_windows):
        pj = jnp.where(km & (pos == j), l64, 0).sum(axis=1, keepdims=True)
        tok_mid = tok_mid + jnp.where(jm == j, pj * omega, 0)
    tok = jnp.where(sl < sink, sl,
                    jnp.where(sl >= mid_end, sl + local_off,
                              tok_mid + sink + rm))
    idx_ref[...] = tok


def _build_index_kernel(h_num, s_len, idx_pad, n_eligible, k_windows,
                        mid_end, local_off):
    body = functools.partial(_index_body, h_num, s_len, idx_pad, n_eligible,
                             k_windows, SINK, OMEGA, mid_end, local_off)
    return pl.pallas_call(
        body,
        out_shape=jax.ShapeDtypeStruct((h_num, idx_pad), jnp.int32),
    )


def _build_sc_gather(h_num, kept_len, s_len, d, idx_pad, ch):
    mesh = plsc.VectorSubcoreMesh(core_axis_name="c", subcore_axis_name="s")
    nw = 32
    n_chunks = (h_num * kept_len) // ch
    per_w = n_chunks // nw
    cph = kept_len // ch

    @functools.partial(
        pl.kernel, mesh=mesh,
        out_type=(jax.ShapeDtypeStruct((h_num, kept_len, d), jnp.float32),
                  jax.ShapeDtypeStruct((h_num, kept_len, d), jnp.float32)),
        scratch_types=[pltpu.VMEM((ch,), jnp.int32),
                       pltpu.VMEM((ch, d), jnp.float32),
                       pltpu.VMEM((ch, d), jnp.float32),
                       pltpu.SemaphoreType.DMA,
                       pltpu.SemaphoreType.DMA],
        compiler_params=pltpu.CompilerParams(use_tc_tiling_on_sc=False),
    )
    def gat(keys_hbm, vals_hbm, idx_hbm, out_k, out_v, idxv, rk, rv, sk, sv):
        wid = lax.axis_index("s") * 2 + lax.axis_index("c")
        for c in range(per_w):
            g = wid * per_w + c
            head = g // cph
            roff = (g % cph) * ch
            pltpu.sync_copy(idx_hbm.at[head, pl.ds(roff, ch)], idxv)
            a = pltpu.async_copy(keys_hbm.at[head].at[idxv], rk, sk)
            b = pltpu.async_copy(vals_hbm.at[head].at[idxv], rv, sv)
            a.wait()
            b.wait()
            pltpu.sync_copy(rk, out_k.at[head, pl.ds(roff, ch)])
            pltpu.sync_copy(rv, out_v.at[head, pl.ds(roff, ch)])

    return gat


def kernel(past_key, past_value, attn_score_cache):
    b, h_num, s_len, d = past_key.shape
    assert b == 1
    local_num = int(P_RATIO * s_len) // OMEGA
    n_win = (s_len - SINK) // OMEGA
    budget_tokens = int(R_RATIO * s_len)
    k_windows = max((budget_tokens - SINK) // OMEGA - 1 - local_num - START_IDX, 1)
    n_eligible = n_win - local_num
    local_start = SINK + n_eligible * OMEGA
    mid_end = SINK + k_windows * OMEGA
    kept_len = mid_end + (s_len - local_start)
    local_off = local_start - mid_end
    assert n_win <= W_PAD
    idx_pad = -(-kept_len // 128) * 128

    attn3 = attn_score_cache.reshape(h_num, s_len, s_len)
    col = _build_reduce_kernel(h_num, s_len)(attn3)
    idx = _build_index_kernel(h_num, s_len, idx_pad, n_eligible, k_windows,
                              mid_end, local_off)(col)

    keys3 = past_key.reshape(h_num, s_len, d)
    vals3 = past_value.reshape(h_num, s_len, d)
    out_k, out_v = _build_sc_gather(h_num, kept_len, s_len, d, idx_pad, 96)(
        keys3, vals3, idx)
    return (out_k.reshape(b, h_num, kept_len, d),
            out_v.reshape(b, h_num, kept_len, d))
